# Initial kernel scaffold; baseline (speedup 1.0000x reference)
import jax
import jax.numpy as jnp
from jax.experimental import pallas as pl


def kernel(x, plane0, plane1, plane2):
    raise NotImplementedError("write your pallas kernel here")


# speedup vs baseline: 49.9373x; 49.9373x over previous
import functools

import jax
import jax.numpy as jnp
from jax import lax
from jax.experimental import pallas as pl
from jax.experimental.pallas import tpu as pltpu
from jax.experimental.pallas import tpu_sc as plsc

_C = 32
_RES = 512
_B = 1000000
_NW = 32
_K = 128
_CHUNKS = 245
_BPAD = _NW * _K * _CHUNKS


def _sc_body(t0, t1, t2, x0h, x1h, x2h, out_h,
             c0v, c1v, c2v, idxv, wv, r0, r1, r2, r3, r4, r5, r6, r7, r8, r9,
             r10, r11, outv, sem):
    tables = (t0, t1, t2)
    rows = (r0, r1, r2, r3, r4, r5, r6, r7, r8, r9, r10, r11)
    wid = lax.axis_index("c") * 16 + lax.axis_index("s")
    wbase = wid * (_K * _CHUNKS)

    def chunk_body(i, _):
        base = wbase + i * _K
        pltpu.sync_copy(x0h.at[pl.ds(base, _K)], c0v)
        pltpu.sync_copy(x1h.at[pl.ds(base, _K)], c1v)
        pltpu.sync_copy(x2h.at[pl.ds(base, _K)], c2v)

        def idxw_body(g, _):
            s = g * 16
            iis = []
            ffs = []
            for cv in (c0v, c1v, c2v):
                c = cv[pl.ds(s, 16)]
                p = c * 255.5 + 255.5
                ii = p.astype(jnp.int32)
                ii = jnp.minimum(jnp.maximum(ii, 0), _RES - 2)
                iis.append(ii)
                ffs.append(p - ii.astype(jnp.float32))
            combos = (
                (iis[0], ffs[0], iis[1], ffs[1]),
                (iis[0], ffs[0], iis[2], ffs[2]),
                (iis[1], ffs[1], iis[2], ffs[2]),
            )
            for pnum, (xi, fx, yi, fy) in enumerate(combos):
                b = yi * _RES + xi
                idxv[4 * pnum + 0, pl.ds(s, 16)] = b
                idxv[4 * pnum + 1, pl.ds(s, 16)] = b + 1
                idxv[4 * pnum + 2, pl.ds(s, 16)] = b + _RES
                idxv[4 * pnum + 3, pl.ds(s, 16)] = b + _RES + 1
                gx = 1.0 - fx
                gy = 1.0 - fy
                wv[4 * pnum + 0, pl.ds(s, 16)] = gy * gx
                wv[4 * pnum + 1, pl.ds(s, 16)] = gy * fx
                wv[4 * pnum + 2, pl.ds(s, 16)] = fy * gx
                wv[4 * pnum + 3, pl.ds(s, 16)] = fy * fx
            return _

        lax.fori_loop(0, _K // 16, idxw_body, None)

        copies = [
            pltpu.async_copy(tables[r // 4].at[idxv.at[r]], rows[r], sem)
            for r in range(12)
        ]
        for cp in copies:
            cp.wait()

        def comb_body(g, _):
            s = g * 16
            ws = [wv[r, pl.ds(s, 16)] for r in range(12)]
            for j in range(16):
                k = s + j
                for half in (0, 16):
                    acc = None
                    for pnum in range(3):
                        t = None
                        for q in range(4):
                            r = 4 * pnum + q
                            v = rows[r][k, pl.ds(half, 16)] * ws[r][j]
                            t = v if t is None else t + v
                        acc = t if acc is None else acc * t
                    outv[k, pl.ds(half, 16)] = acc
            return _

        lax.fori_loop(0, _K // 16, comb_body, None)
        pltpu.sync_copy(outv, out_h.at[pl.ds(base, _K)])
        return _

    lax.fori_loop(0, _CHUNKS, chunk_body, None)


@jax.jit
def kernel(x, plane0, plane1, plane2):
    tables = [
        jnp.transpose(p[0], (1, 2, 0)).reshape(_RES * _RES, _C)
        for p in (plane0, plane1, plane2)
    ]
    xp = jnp.pad(x, ((0, _BPAD - _B), (0, 0)))
    cols = [xp[:, j] for j in range(3)]

    mesh = plsc.VectorSubcoreMesh(
        core_axis_name="c", subcore_axis_name="s", num_cores=2, num_subcores=16
    )
    scratch = [
        pltpu.VMEM((_K,), jnp.float32),
        pltpu.VMEM((_K,), jnp.float32),
        pltpu.VMEM((_K,), jnp.float32),
        pltpu.VMEM((12, _K), jnp.int32),
        pltpu.VMEM((12, _K), jnp.float32),
    ] + [pltpu.VMEM((_K, _C), jnp.float32) for _ in range(12)] + [
        pltpu.VMEM((_K, _C), jnp.float32),
        pltpu.SemaphoreType.DMA,
    ]
    out = pl.kernel(
        _sc_body,
        out_type=jax.ShapeDtypeStruct((_BPAD, _C), jnp.float32),
        mesh=mesh,
        scratch_types=scratch,
        compiler_params=pltpu.CompilerParams(use_tc_tiling_on_sc=False),
    )(*tables, *cols)
    return out[:_B]


# speedup vs baseline: 65.3935x; 1.3095x over previous
import jax
import jax.numpy as jnp
from jax import lax
from jax.experimental import pallas as pl
from jax.experimental.pallas import tpu as pltpu
from jax.experimental.pallas import tpu_sc as plsc

_C = 32
_RES = 512
_B = 1000000
_NW = 32
_K = 128
_CHUNKS = 246
_BPAD = _NW * _K * _CHUNKS
_CPAD = _BPAD + 2 * _K
_PLANE_OFF = _RES * _RES


def _sc_body(table, x0h, x1h, x2h, out_h,
             cb0, cb1, idx0, idx1, w0, w1, rows0, rows1,
             outv, sg0, sg1, sc0, sc1):
    xs = (x0h, x1h, x2h)
    rows = (rows0, rows1)
    cbufs = (cb0, cb1)
    idxs = (idx0, idx1)
    wvs = (w0, w1)
    gsems = (sg0, sg1)
    csems = (sc0, sc1)
    wid = lax.axis_index("c") * 16 + lax.axis_index("s")
    wbase = wid * (_K * _CHUNKS)

    def fire_coords(j, par):
        base = wbase + j * _K
        for a in range(3):
            pltpu.async_copy(xs[a].at[pl.ds(base, _K)], cbufs[par].at[a], csems[par])

    def wait_coords(par):
        for a in range(3):
            pltpu.make_async_copy(xs[a].at[pl.ds(0, _K)], cbufs[par].at[a],
                                  csems[par]).wait()

    def idx_pass(par):
        cb, idxv, wv = cbufs[par], idxs[par], wvs[par]

        def body(g, carry):
            s = g * 16
            iis = []
            ffs = []
            for a in range(3):
                c = cb[a, pl.ds(s, 16)]
                p = c * 255.5 + 255.5
                ii = p.astype(jnp.int32)
                ii = jnp.minimum(jnp.maximum(ii, 0), _RES - 2)
                iis.append(ii)
                ffs.append(p - ii.astype(jnp.float32))
            combos = (
                (iis[0], ffs[0], iis[1], ffs[1]),
                (iis[0], ffs[0], iis[2], ffs[2]),
                (iis[1], ffs[1], iis[2], ffs[2]),
            )
            for pnum, (xi, fx, yi, fy) in enumerate(combos):
                b = yi * _RES + xi + pnum * _PLANE_OFF
                idxv[4 * pnum + 0, pl.ds(s, 16)] = b
                idxv[4 * pnum + 1, pl.ds(s, 16)] = b + 1
                idxv[4 * pnum + 2, pl.ds(s, 16)] = b + _RES
                idxv[4 * pnum + 3, pl.ds(s, 16)] = b + _RES + 1
                gx = 1.0 - fx
                gy = 1.0 - fy
                wv[4 * pnum + 0, pl.ds(s, 16)] = gy * gx
                wv[4 * pnum + 1, pl.ds(s, 16)] = gy * fx
                wv[4 * pnum + 2, pl.ds(s, 16)] = fy * gx
                wv[4 * pnum + 3, pl.ds(s, 16)] = fy * fx
            return carry

        lax.fori_loop(0, _K // 16, body, None)

    def fire_gathers(par):
        for r in range(12):
            pltpu.async_copy(table.at[idxs[par].at[r]], rows[par].at[r],
                             gsems[par])

    def wait_gathers(par):
        for r in range(12):
            pltpu.make_async_copy(table.at[idxs[par].at[r]], rows[par].at[r],
                                  gsems[par]).wait()

    def combine_store(j, par):
        wv = wvs[par]
        rw = rows[par]

        def body(g, carry):
            s = g * 16
            ws = [wv[r, pl.ds(s, 16)] for r in range(12)]
            for jj in range(16):
                k = s + jj
                for half in (0, 16):
                    acc = None
                    for pnum in range(3):
                        t = None
                        for q in range(4):
                            r = 4 * pnum + q
                            v = rw[r, k, pl.ds(half, 16)] * ws[r][jj]
                            t = v if t is None else t + v
                        acc = t if acc is None else acc * t
                    outv[k, pl.ds(half, 16)] = acc
            return carry

        lax.fori_loop(0, _K // 16, body, None)
        pltpu.sync_copy(outv, out_h.at[pl.ds(wbase + j * _K, _K)])

    fire_coords(0, 0)
    wait_coords(0)
    idx_pass(0)
    fire_gathers(0)
    fire_coords(1, 1)

    def pair_body(t, carry):
        j0 = 2 * t
        wait_coords(1)
        idx_pass(1)
        fire_gathers(1)
        fire_coords(j0 + 2, 0)
        wait_gathers(0)
        combine_store(j0, 0)
        wait_coords(0)
        idx_pass(0)
        fire_gathers(0)
        fire_coords(j0 + 3, 1)
        wait_gathers(1)
        combine_store(j0 + 1, 1)
        return carry

    lax.fori_loop(0, _CHUNKS // 2, pair_body, None)

    wait_gathers(0)
    wait_coords(1)


@jax.jit
def kernel(x, plane0, plane1, plane2):
    table = jnp.concatenate(
        [
            jnp.transpose(p[0], (1, 2, 0)).reshape(_RES * _RES, _C)
            for p in (plane0, plane1, plane2)
        ],
        axis=0,
    )
    xp = jnp.pad(x, ((0, _CPAD - _B), (0, 0)))
    cols = [xp[:, j] for j in range(3)]

    mesh = plsc.VectorSubcoreMesh(
        core_axis_name="c", subcore_axis_name="s", num_cores=2, num_subcores=16
    )
    scratch = [
        pltpu.VMEM((3, _K), jnp.float32),
        pltpu.VMEM((3, _K), jnp.float32),
        pltpu.VMEM((12, _K), jnp.int32),
        pltpu.VMEM((12, _K), jnp.int32),
        pltpu.VMEM((12, _K), jnp.float32),
        pltpu.VMEM((12, _K), jnp.float32),
        pltpu.VMEM((12, _K, _C), jnp.float32),
        pltpu.VMEM((12, _K, _C), jnp.float32),
        pltpu.VMEM((_K, _C), jnp.float32),
        pltpu.SemaphoreType.DMA,
        pltpu.SemaphoreType.DMA,
        pltpu.SemaphoreType.DMA,
        pltpu.SemaphoreType.DMA,
    ]
    out = pl.kernel(
        _sc_body,
        out_type=jax.ShapeDtypeStruct((_BPAD, _C), jnp.float32),
        mesh=mesh,
        scratch_types=scratch,
        compiler_params=pltpu.CompilerParams(use_tc_tiling_on_sc=False),
    )(table, *cols)
    return out[:_B]
